# baseline (speedup 1.0000x reference)
import jax
import jax.numpy as jnp
from jax.experimental import pallas as pl
from jax.experimental.pallas import tpu as pltpu


def _copy_kernel(src_ref, dst_ref, sem):
    cp = pltpu.make_async_copy(src_ref, dst_ref, sem)
    cp.start()
    cp.wait()


def kernel(flat_values, cu_seqlens):
    del cu_seqlens
    return pl.pallas_call(
        _copy_kernel,
        out_shape=jax.ShapeDtypeStruct(flat_values.shape, flat_values.dtype),
        in_specs=[pl.BlockSpec(memory_space=pl.ANY)],
        out_specs=pl.BlockSpec(memory_space=pl.ANY),
        scratch_shapes=[pltpu.SemaphoreType.DMA],
    )(flat_values)
